# baseline (speedup 1.0000x reference)
import functools

import jax
import jax.numpy as jnp
from jax.experimental import pallas as pl
from jax.experimental.pallas import tpu as pltpu

H = 128
NB = 64
N_GE, E_GE = 40704, 325632
N_CNV, E_CNV = 44416, 355328
N_MUT, E_MUT = 40704, 325632

_BLK = 512


def _mm_body(x_ref, w_ref, b_ref, o_ref):
    o_ref[...] = jnp.dot(x_ref[...], w_ref[...],
                         preferred_element_type=jnp.float32) + b_ref[...]


def matmul_bias(x, w, b):
    n = x.shape[0]
    blk = _BLK
    while n % blk:
        blk //= 2
    grid = (n // blk,)
    return pl.pallas_call(
        _mm_body,
        grid=grid,
        in_specs=[
            pl.BlockSpec((blk, H), lambda i: (i, 0)),
            pl.BlockSpec((H, H), lambda i: (0, 0)),
            pl.BlockSpec((H,), lambda i: (0,)),
        ],
        out_specs=pl.BlockSpec((blk, H), lambda i: (i, 0)),
        out_shape=jax.ShapeDtypeStruct((n, H), jnp.float32),
    )(x, w, b.reshape(1, H)[0])


def gcn_norm_loops(edge_index, n):
    loop = jnp.arange(n, dtype=edge_index.dtype)
    src = jnp.concatenate([edge_index[0], loop])
    dst = jnp.concatenate([edge_index[1], loop])
    deg = jax.ops.segment_sum(jnp.ones(src.shape[0], jnp.float32), dst,
                              num_segments=n)
    dinv = jnp.where(deg > 0, 1.0 / jnp.sqrt(deg), 0.0)
    return src, dst, dinv[src] * dinv[dst]


def gcn_conv(x, src, dst, norm, W, b, n):
    h = matmul_bias(x, W, jnp.zeros((H,), jnp.float32))
    return jax.ops.segment_sum(h[src] * norm[:, None], dst,
                               num_segments=n) + b


def fa_conv(x, x0, src, dst, norm, att_l, att_r, n, eps=0.1):
    al = x @ att_l[:, None]
    ar = x @ att_r[:, None]
    alpha = jnp.tanh(al[src] + ar[dst])
    out = jax.ops.segment_sum(x[src] * alpha * norm[:, None], dst,
                              num_segments=n)
    return out + eps * x0


def cell_encoder(x_idx, edge_index, batch, embed, att_l, att_r, n):
    x = embed[x_idx]
    src, dst, norm = gcn_norm_loops(edge_index, n)
    x1 = fa_conv(x, x, src, dst, norm, att_l[0], att_r[0], n)
    x2 = fa_conv(x1, x, src, dst, norm, att_l[1], att_r[1], n)
    x3 = fa_conv(x2, x, src, dst, norm, att_l[2], att_r[2], n)
    sums = jax.ops.segment_sum(x3, batch, num_segments=NB)
    cnt = jax.ops.segment_sum(jnp.ones(n, jnp.float32), batch,
                              num_segments=NB)
    return sums / jnp.maximum(cnt, 1.0)[:, None]


def sim_gnn(x, ei, ei_sim, batch, lw, lb, gw, gb, sw, sb, ww, wb, n):
    hidden = x @ lw + lb
    s1, d1, n1 = gcn_norm_loops(ei, n)
    s2, d2, n2 = gcn_norm_loops(ei_sim, n)
    for i in range(3):
        xg = jax.nn.relu(gcn_conv(hidden, s1, d1, n1, gw[i], gb[i], n))
        xs = jax.nn.relu(gcn_conv(hidden, s2, d2, n2, sw[i], sb[i], n))
        s = jax.nn.sigmoid(hidden @ ww[i][:, None] + wb[i])
        hidden = s * xg + (1.0 - s) * xs
    return jax.ops.segment_max(hidden, batch, num_segments=NB)


def kernel(ge_x, ge_edge_index, ge_sim_edge_index, ge_batch, cnv_x,
           cnv_edge_index, cnv_batch, mut_x, mut_edge_index, mut_batch,
           mut_embed, mut_att_l, mut_att_r, cnv_embed, cnv_att_l, cnv_att_r,
           ge_lin1_w, ge_lin1_b, ge_gcn_w, ge_gcn_b, ge_sim_w, ge_sim_b,
           ge_wt_w, ge_wt_b):
    mut_repr = cell_encoder(mut_x, mut_edge_index, mut_batch, mut_embed,
                            mut_att_l, mut_att_r, N_MUT)
    cnv_repr = cell_encoder(cnv_x, cnv_edge_index, cnv_batch, cnv_embed,
                            cnv_att_l, cnv_att_r, N_CNV)
    ge_repr = sim_gnn(ge_x, ge_edge_index, ge_sim_edge_index, ge_batch,
                      ge_lin1_w, ge_lin1_b, ge_gcn_w, ge_gcn_b, ge_sim_w,
                      ge_sim_b, ge_wt_w, ge_wt_b, N_GE)
    return (mut_repr, cnv_repr, ge_repr)


# speedup vs baseline: 1.2540x; 1.2540x over previous
import functools

import jax
import jax.numpy as jnp
from jax import lax
from jax.experimental import pallas as pl
from jax.experimental.pallas import tpu as pltpu
from jax.experimental.pallas import tpu_sc as plsc

H = 128
NB = 64
N_GE, E_GE = 40704, 325632
N_CNV, E_CNV = 44416, 355328
N_MUT, E_MUT = 40704, 325632

NC, NS, LANES = 2, 16, 16
SCH = 512
NSUB = SCH // 128
EPAD_UNIT = NS * SCH


def _mm_body(x_ref, w_ref, b_ref, s_ref, o_ref):
    o_ref[...] = (jnp.dot(x_ref[...], w_ref[...],
                          preferred_element_type=jnp.float32)
                  + b_ref[...]) * s_ref[...]


def matmul_bias(x, w, b, rowscale=None):
    n = x.shape[0]
    blk = 512
    while n % blk:
        blk //= 2
    if rowscale is None:
        rowscale = jnp.ones((n, 1), jnp.float32)
    return pl.pallas_call(
        _mm_body,
        grid=(n // blk,),
        in_specs=[
            pl.BlockSpec((blk, H), lambda i: (i, 0)),
            pl.BlockSpec((H, H), lambda i: (0, 0)),
            pl.BlockSpec((H,), lambda i: (0,)),
            pl.BlockSpec((blk, 1), lambda i: (i, 0)),
        ],
        out_specs=pl.BlockSpec((blk, H), lambda i: (i, 0)),
        out_shape=jax.ShapeDtypeStruct((n, H), jnp.float32),
    )(x, w, b.reshape(1, H)[0], rowscale)


def _edge_body(n, e_pad, scale,
               xs, src2, dst2, cexp2, out,
               idx_s, idx_d, cexp, rows, zbuf, accum,
               gsem, ssem):
    c = lax.axis_index("c")
    s = lax.axis_index("s")
    npt = n // NS
    full = npt // 128
    rem = npt % 128
    nsc = e_pad // (NS * SCH)
    row0 = s * npt
    zv = jnp.zeros((LANES,), jnp.float32)

    for r in range(128):
        for j in range(2):
            zbuf[r, pl.ds(j * LANES, LANES)] = zv

    for qi in range(2):
        qoff = (2 * c + qi) * n

        for t in range(full):
            pltpu.sync_copy(zbuf, accum.at[pl.ds(row0 + t * 128, 128), :])
        if rem:
            pltpu.sync_copy(zbuf.at[pl.ds(0, rem), :],
                            accum.at[pl.ds(row0 + full * 128, rem), :])
        plsc.subcore_barrier()

        @pl.loop(0, nsc)
        def sc_loop(k):
            rb = (s * nsc + k) * NSUB
            eb = (s * nsc + k) * SCH
            pltpu.sync_copy(src2.at[pl.ds(rb, NSUB), :], idx_s)
            pltpu.sync_copy(dst2.at[pl.ds(rb, NSUB), :], idx_d)
            if scale:
                pltpu.sync_copy(cexp2.at[pl.ds(eb, SCH), :], cexp)

            @pl.loop(0, NSUB)
            def adj(i):
                for j in range(8):
                    sl = pl.ds(j * LANES, LANES)
                    idx_s[i, sl] = idx_s[i, sl] + qoff

            gds = [pltpu.async_copy(xs.at[idx_s.at[i]],
                                    rows.at[pl.ds(i * 128, 128), :], gsem)
                   for i in range(NSUB)]
            for d in gds:
                d.wait()

            if scale:
                @pl.loop(0, SCH, unroll=8)
                def scale_e(e):
                    cv = cexp[e, pl.ds(0, LANES)]
                    for j in range(2):
                        sl = pl.ds(j * LANES, LANES)
                        rows[e, sl] = rows[e, sl] * cv

            sds = [pltpu.async_copy(rows.at[pl.ds(i * 128, 128), :],
                                    accum.at[idx_d.at[i]], ssem, add=True)
                   for i in range(NSUB)]
            for d in sds:
                d.wait()

        plsc.subcore_barrier()

        for t in range(full):
            pltpu.sync_copy(accum.at[pl.ds(row0 + t * 128, 128), :],
                            out.at[pl.ds(qoff + row0 + t * 128, 128), :])
        if rem:
            pltpu.sync_copy(accum.at[pl.ds(row0 + full * 128, rem), :],
                            out.at[pl.ds(qoff + row0 + full * 128, rem), :])
        if qi == 0:
            plsc.subcore_barrier()


@functools.lru_cache(maxsize=None)
def _make_edge_pass(n, e_pad, scale):
    mesh = plsc.VectorSubcoreMesh(core_axis_name="c", subcore_axis_name="s")
    return pl.kernel(
        functools.partial(_edge_body, n, e_pad, scale),
        out_type=jax.ShapeDtypeStruct((4 * n, 32), jnp.float32),
        mesh=mesh,
        compiler_params=pltpu.CompilerParams(use_tc_tiling_on_sc=False),
        scratch_types=[
            pltpu.VMEM((NSUB, 128), jnp.int32),
            pltpu.VMEM((NSUB, 128), jnp.int32),
            pltpu.VMEM((SCH, LANES), jnp.float32),
            pltpu.VMEM((SCH, 32), jnp.float32),
            pltpu.VMEM((128, 32), jnp.float32),
            pltpu.VMEM_SHARED((n, 32), jnp.float32),
            pltpu.SemaphoreType.DMA,
            pltpu.SemaphoreType.DMA,
        ],
    )


def _to_stack(x):
    n = x.shape[0]
    return x.reshape(n, 4, 32).transpose(1, 0, 2).reshape(4 * n, 32)


def _from_stack(y, n):
    return y.reshape(4, n, 32).transpose(1, 0, 2).reshape(n, H)


def _pad_to(a, m, fill=0):
    pad = m - a.shape[0]
    if pad == 0:
        return a
    return jnp.concatenate([a, jnp.full((pad,), fill, a.dtype)])


def edge_aggregate(x, src_p, dst_p, coef_p, n, e_pad):
    scale = coef_p is not None
    f = _make_edge_pass(n, e_pad, scale)
    src2 = src_p.reshape(e_pad // 128, 128)
    dst2 = dst_p.reshape(e_pad // 128, 128)
    if scale:
        cexp2 = jnp.broadcast_to(coef_p[:, None], (e_pad, LANES))
    else:
        cexp2 = jnp.zeros((1, LANES), jnp.float32)
    y = f(_to_stack(x), src2, dst2, cexp2)
    return _from_stack(y, n)


def _round_up(e):
    return ((e + EPAD_UNIT - 1) // EPAD_UNIT) * EPAD_UNIT


def gcn_norm_loops(edge_index, n):
    loop = jnp.arange(n, dtype=edge_index.dtype)
    src = jnp.concatenate([edge_index[0], loop])
    dst = jnp.concatenate([edge_index[1], loop])
    deg = jax.ops.segment_sum(jnp.ones(src.shape[0], jnp.float32), dst,
                              num_segments=n)
    dinv = jnp.where(deg > 0, 1.0 / jnp.sqrt(deg), 0.0)
    return src, dst, dinv


def cell_encoder(x_idx, edge_index, batch, embed, att_l, att_r, n):
    x = embed[x_idx]
    src, dst, dinv = gcn_norm_loops(edge_index, n)
    norm = dinv[src] * dinv[dst]
    e_tot = src.shape[0]
    e_pad = _round_up(e_tot)
    src_p = _pad_to(src, e_pad)
    dst_p = _pad_to(dst, e_pad)

    def fa(xc, x0, att_l_i, att_r_i):
        al = xc @ att_l_i
        ar = xc @ att_r_i
        alpha = jnp.tanh(al[src] + ar[dst])
        coef_p = _pad_to(norm * alpha, e_pad)
        out = edge_aggregate(xc, src_p, dst_p, coef_p, n, e_pad)
        return out + 0.1 * x0

    x1 = fa(x, x, att_l[0], att_r[0])
    x2 = fa(x1, x, att_l[1], att_r[1])
    x3 = fa(x2, x, att_l[2], att_r[2])
    sums = jax.ops.segment_sum(x3, batch, num_segments=NB)
    cnt = jax.ops.segment_sum(jnp.ones(n, jnp.float32), batch,
                              num_segments=NB)
    return sums / jnp.maximum(cnt, 1.0)[:, None]


def sim_gnn(x, ei, ei_sim, batch, lw, lb, gw, gb, sw, sb, ww, wb, n):
    hidden = x @ lw + lb
    e_pad = _round_up(ei.shape[1])
    sets = []
    for e in (ei, ei_sim):
        _, _, dinv = gcn_norm_loops(e, n)
        sets.append((_pad_to(e[0], e_pad), _pad_to(e[1], e_pad),
                     dinv.reshape(n, 1)))

    npad = e_pad - ei.shape[1]

    def gconv(hc, W, b, sp, dp, dinv):
        hp = matmul_bias(hc, W, jnp.zeros((H,), jnp.float32), rowscale=dinv)
        agg = edge_aggregate(hp, sp, dp, None, n, e_pad)
        agg = agg.at[0].add(-float(npad) * hp[0])
        return dinv * (agg + hp) + b

    for i in range(3):
        xg = jax.nn.relu(gconv(hidden, gw[i], gb[i], *sets[0]))
        xs = jax.nn.relu(gconv(hidden, sw[i], sb[i], *sets[1]))
        s = jax.nn.sigmoid(hidden @ ww[i][:, None] + wb[i])
        hidden = s * xg + (1.0 - s) * xs
    return jax.ops.segment_max(hidden, batch, num_segments=NB)


def kernel(ge_x, ge_edge_index, ge_sim_edge_index, ge_batch, cnv_x,
           cnv_edge_index, cnv_batch, mut_x, mut_edge_index, mut_batch,
           mut_embed, mut_att_l, mut_att_r, cnv_embed, cnv_att_l, cnv_att_r,
           ge_lin1_w, ge_lin1_b, ge_gcn_w, ge_gcn_b, ge_sim_w, ge_sim_b,
           ge_wt_w, ge_wt_b):
    mut_repr = cell_encoder(mut_x, mut_edge_index, mut_batch, mut_embed,
                            mut_att_l, mut_att_r, N_MUT)
    cnv_repr = cell_encoder(cnv_x, cnv_edge_index, cnv_batch, cnv_embed,
                            cnv_att_l, cnv_att_r, N_CNV)
    ge_repr = sim_gnn(ge_x, ge_edge_index, ge_sim_edge_index, ge_batch,
                      ge_lin1_w, ge_lin1_b, ge_gcn_w, ge_gcn_b, ge_sim_w,
                      ge_sim_b, ge_wt_w, ge_wt_b, N_GE)
    return (mut_repr, cnv_repr, ge_repr)
